# Initial kernel scaffold; baseline (speedup 1.0000x reference)
import jax
import jax.numpy as jnp
from jax.experimental import pallas as pl


def kernel(x, edge_index, batch, target, sgc_W, sgc_b, fcg1_W, fcg1_b, emb, conv_W, conv_b, fc1xt_W, fc1xt_b, fc1_W, fc1_b, fc2_W, fc2_b, out_W, out_b):
    raise NotImplementedError("write your pallas kernel here")


# speedup vs baseline: 1.2362x; 1.2362x over previous
import jax
import jax.numpy as jnp
from jax.experimental import pallas as pl

_N = 50000
_E = 800000
_NG = 256
_FXD = 78
_HID = 312
_OUT_DIM = 128
_EMB = 128
_NF = 8
_KS = 8
_CONV_OUT = _EMB - _KS + 1
_XT_FLAT = _NF * _CONV_OUT
_SEQ = 1000
_VOC = 26

_NODE_BLK = 2000
_G_BLK = 8


def _sgc_linear_kernel(h_ref, w_ref, b_ref, o_ref):
    acc = jax.lax.dot_general(
        h_ref[...], w_ref[...], (((1,), (1,)), ((), ())),
        preferred_element_type=jnp.float32)
    acc = acc + b_ref[...][None, :]
    o_ref[...] = jnp.where(acc > 0, acc, 0.01 * acc)


def _conv_kernel(et_ref, convw_ref, convb_ref, o_ref):
    convw = convw_ref[...]
    etb = et_ref[...]
    for n in range(_G_BLK):
        etn = etb[n]
        acc = jnp.zeros((_NF, _CONV_OUT), jnp.float32)
        for k in range(_KS):
            acc = acc + jax.lax.dot_general(
                convw[:, :, k], etn[:, k:k + _CONV_OUT],
                (((1,), (0,)), ((), ())),
                preferred_element_type=jnp.float32)
        o_ref[n] = acc + convb_ref[...][:, None]


def _head_kernel(g_ref, xt_ref, fcg1w_ref, fcg1b_ref, fxw_ref, fxb_ref,
                 f1w_ref, f1b_ref, f2w_ref, f2b_ref, ow_ref, ob_ref, o_ref):
    g = jax.lax.dot_general(
        g_ref[...], fcg1w_ref[...], (((1,), (1,)), ((), ())),
        preferred_element_type=jnp.float32) + fcg1b_ref[...][None, :]
    g = jnp.where(g > 0, g, 0.01 * g)
    xt = jax.lax.dot_general(
        xt_ref[...], fxw_ref[...], (((1,), (1,)), ((), ())),
        preferred_element_type=jnp.float32) + fxb_ref[...][None, :]
    xc = jnp.concatenate([g, xt], axis=1)
    xc = jax.lax.dot_general(
        xc, f1w_ref[...], (((1,), (1,)), ((), ())),
        preferred_element_type=jnp.float32) + f1b_ref[...][None, :]
    xc = jnp.maximum(xc, 0.0)
    xc = jax.lax.dot_general(
        xc, f2w_ref[...], (((1,), (1,)), ((), ())),
        preferred_element_type=jnp.float32) + f2b_ref[...][None, :]
    xc = jnp.maximum(xc, 0.0)
    o_ref[...] = jnp.sum(xc * ow_ref[...], axis=1, keepdims=True) + ob_ref[...][None, :]


def kernel(x, edge_index, batch, target, sgc_W, sgc_b, fcg1_W, fcg1_b, emb,
           conv_W, conv_b, fc1xt_W, fc1xt_b, fc1_W, fc1_b, fc2_W, fc2_b,
           out_W, out_b):
    row, col = edge_index[0], edge_index[1]
    deg = jnp.ones((_N,), jnp.float32).at[col].add(
        jnp.ones((_E,), jnp.float32))
    dinv = jax.lax.rsqrt(deg)
    norm_e = dinv[row] * dinv[col]
    norm_l = dinv * dinv
    h = x
    for _ in range(2):
        msg = h[row] * norm_e[:, None]
        h = (h * norm_l[:, None]).at[col].add(msg)

    h = pl.pallas_call(
        _sgc_linear_kernel,
        grid=(_N // _NODE_BLK,),
        in_specs=[
            pl.BlockSpec((_NODE_BLK, _FXD), lambda i: (i, 0)),
            pl.BlockSpec((_HID, _FXD), lambda i: (0, 0)),
            pl.BlockSpec((_HID,), lambda i: (0,)),
        ],
        out_specs=pl.BlockSpec((_NODE_BLK, _HID), lambda i: (i, 0)),
        out_shape=jax.ShapeDtypeStruct((_N, _HID), jnp.float32),
    )(h, sgc_W, sgc_b)

    g = jax.ops.segment_max(h, batch, num_segments=_NG)

    et = emb[target]
    conv = pl.pallas_call(
        _conv_kernel,
        grid=(_NG // _G_BLK,),
        in_specs=[
            pl.BlockSpec((_G_BLK, _SEQ, _EMB), lambda i: (i, 0, 0)),
            pl.BlockSpec((_NF, _SEQ, _KS), lambda i: (0, 0, 0)),
            pl.BlockSpec((_NF,), lambda i: (0,)),
        ],
        out_specs=pl.BlockSpec((_G_BLK, _NF, _CONV_OUT), lambda i: (i, 0, 0)),
        out_shape=jax.ShapeDtypeStruct((_NG, _NF, _CONV_OUT), jnp.float32),
    )(et, conv_W, conv_b)
    xt_flat = conv.reshape(_NG, _XT_FLAT)

    out = pl.pallas_call(
        _head_kernel,
        out_shape=jax.ShapeDtypeStruct((_NG, 1), jnp.float32),
    )(g, xt_flat, fcg1_W, fcg1_b, fc1xt_W, fc1xt_b,
      fc1_W, fc1_b, fc2_W, fc2_b, out_W, out_b)
    return out
